# baseline (speedup 1.0000x reference)
import functools

import jax
import jax.numpy as jnp
from jax import lax
from jax.experimental import pallas as pl
from jax.experimental.pallas import tpu as pltpu
from jax.experimental.pallas import tpu_sc as plsc

N = 10000
E = 320000
C_IN = 128
C_HID = 256
C_OUT = 128

NC = 2
NT = 16
NW = NC * NT

_MESH = plsc.VectorSubcoreMesh(core_axis_name="c", subcore_axis_name="s")


EPW = E // NW


@functools.partial(
    pl.kernel, mesh=_MESH,
    out_type=jax.ShapeDtypeStruct((NW * N,), jnp.float32),
    scratch_types=[
        pltpu.VMEM((EPW,), jnp.int32),
        pltpu.VMEM((N,), jnp.float32),
    ],
    compiler_params=pltpu.CompilerParams(needs_layout_passes=False),
)
def _deg_kernel(col_hbm, out_hbm, colv, degv):
    wid = lax.axis_index("s") * NC + lax.axis_index("c")
    pltpu.sync_copy(col_hbm.at[pl.ds(wid * EPW, EPW)], colv)
    zeros = jnp.zeros((16,), jnp.float32)

    def _zero(i, carry):
        degv[pl.ds(i * 16, 16)] = zeros
        return carry

    lax.fori_loop(0, N // 16, _zero, 0)
    ones = jnp.ones((16,), jnp.float32)

    def _count(i, carry):
        idx = colv[pl.ds(i * 16, 16)]
        plsc.addupdate_scatter(degv, [idx], ones)
        return carry

    lax.fori_loop(0, EPW // 16, _count, 0)
    pltpu.sync_copy(degv, out_hbm.at[pl.ds(wid * N, N)])


def _make_scatter(H, K, NBUF, edge_split):
    EPT = (E // NC // NT) if edge_split else (E // NT)
    NCH = EPT // K
    RPT = 624
    TAIL0 = NT * RPT
    TAILN = N - TAIL0
    assert NCH % NBUF == 0 and K % 8 == 0 and K <= 128

    @functools.partial(
        pl.kernel, mesh=_MESH,
        out_type=jax.ShapeDtypeStruct((NC, N, H), jnp.float32),
        scratch_types=[
            pltpu.VMEM((NBUF, 2, K), jnp.int32),
            pltpu.VMEM((NBUF, K, H), jnp.float32),
            pltpu.VMEM_SHARED((N, H), jnp.float32),
        ] + [pltpu.SemaphoreType.DMA] * (3 * NBUF),
        compiler_params=pltpu.CompilerParams(needs_layout_passes=False),
    )
    def _scatter(g_hbm, idx_hbm, out_hbm,
                 rcv, rows, acc,
                 is0, is1, is2, is3, is4,
                 gs0, gs1, gs2, gs3, gs4,
                 ss0, ss1, ss2, ss3, ss4):
        isem = (is0, is1, is2, is3, is4)
        gsem = (gs0, gs1, gs2, gs3, gs4)
        ssem = (ss0, ss1, ss2, ss3, ss4)
        sc = lax.axis_index("c")
        tid = lax.axis_index("s")
        r0 = tid * RPT
        gb = 0 if edge_split else sc * N
        xb = (sc * NT + tid) * NCH

        def _i_start(c, b):
            pltpu.async_copy(idx_hbm.at[xb + c], rcv.at[b], isem[b])

        def _i_wait(c, b):
            pltpu.make_async_copy(idx_hbm.at[xb + c], rcv.at[b],
                                  isem[b]).wait()

        def _g_start(b):
            pltpu.async_copy(g_hbm.at[rcv.at[b, 0]], rows.at[b], gsem[b])

        def _g_wait(b):
            pltpu.make_async_copy(g_hbm.at[rcv.at[b, 0]], rows.at[b],
                                  gsem[b]).wait()

        def _s_start(b):
            pltpu.async_copy(rows.at[b], acc.at[rcv.at[b, 1]], ssem[b],
                             add=True)

        def _s_wait(b):
            pltpu.make_async_copy(rows.at[b], acc.at[rcv.at[b, 1]],
                                  ssem[b]).wait()

        pltpu.sync_copy(g_hbm.at[pl.ds(gb + r0, RPT)], acc.at[pl.ds(r0, RPT)])


        @pl.when(tid == NT - 1)
        def _init_tail():
            pltpu.sync_copy(g_hbm.at[pl.ds(gb + TAIL0, TAILN)],
                            acc.at[pl.ds(TAIL0, TAILN)])

        plsc.subcore_barrier()
        _i_start(0, 0)

        def _group(p, carry):
            for b in range(NBUF):
                c = p * NBUF + b
                b1 = (b + 1) % NBUF
                bp = (b - 1) % NBUF
                _i_wait(c, b)
                _g_start(b)

                @pl.when(c >= NBUF - 1)
                def _free_next():
                    _s_wait(b1)

                @pl.when(c + 1 < NCH)
                def _prefetch():
                    _i_start(c + 1, b1)

                @pl.when(c >= 1)
                def _drain_prev():
                    _g_wait(bp)
                    _s_start(bp)
            return carry

        lax.fori_loop(0, NCH // NBUF, _group, 0)
        lastb = (NCH - 1) % NBUF
        _g_wait(lastb)
        _s_start(lastb)
        for b in range(1, NBUF):
            _s_wait(b)
        plsc.subcore_barrier()
        pltpu.sync_copy(acc.at[pl.ds(r0, RPT)], out_hbm.at[sc, pl.ds(r0, RPT)])

        @pl.when(tid == NT - 1)
        def _out_tail():
            pltpu.sync_copy(acc.at[pl.ds(TAIL0, TAILN)],
                            out_hbm.at[sc, pl.ds(TAIL0, TAILN)])

    return _scatter


_K = 40
_NCH1 = E // NT // _K
_NCH2 = E // NC // NT // _K
_scatter_l1 = _make_scatter(C_HID // 2, _K, 5, False)
_scatter_l2 = _make_scatter(C_OUT, _K, 5, True)


def _chunk_desc_l1(row, col):
    rowr = jnp.stack([row, row + N]).reshape(NC, NT, _NCH1, _K)
    colr = jnp.broadcast_to(col.reshape(1, NT, _NCH1, _K),
                            (NC, NT, _NCH1, _K))
    return jnp.stack([rowr, colr], axis=3).reshape(-1, 2, _K)


def _chunk_desc_l2(row, col):
    rowr = row.reshape(NC, NT, _NCH2, _K)
    colr = col.reshape(NC, NT, _NCH2, _K)
    return jnp.stack([rowr, colr], axis=3).reshape(-1, 2, _K)


_BM = 1000


def _tcdis_body(degp_ref, dis_ref):
    deg = jnp.sum(degp_ref[...], axis=0) + 1.0
    dis_ref[...] = lax.rsqrt(deg).reshape(-1, 1)


def _tcdis(degp):
    return pl.pallas_call(
        _tcdis_body,
        grid=(1,),
        in_specs=[pl.BlockSpec((NW, N), lambda i: (0, 0))],
        out_specs=pl.BlockSpec((N, 1), lambda i: (0, 0)),
        out_shape=jax.ShapeDtypeStruct((N, 1), jnp.float32),
    )(degp)


def _tc1_body(x_ref, w_ref, b_ref, dis_ref, g_ref):
    dis = dis_ref[...]
    h = lax.dot_general(x_ref[...], w_ref[...], (((1,), (1,)), ((), ())),
                        preferred_element_type=jnp.float32) + b_ref[...]
    g = h * dis
    g_ref[0] = g[:, :C_HID // 2]
    g_ref[1] = g[:, C_HID // 2:]


def _tc1(x, W1, b1r, dis):
    return pl.pallas_call(
        _tc1_body,
        grid=(N // _BM,),
        in_specs=[
            pl.BlockSpec((_BM, C_IN), lambda i: (i, 0)),
            pl.BlockSpec((C_HID, C_IN), lambda i: (0, 0)),
            pl.BlockSpec((1, C_HID), lambda i: (0, 0)),
            pl.BlockSpec((_BM, 1), lambda i: (i, 0)),
        ],
        out_specs=pl.BlockSpec((NC, _BM, C_HID // 2), lambda i: (0, i, 0)),
        out_shape=jax.ShapeDtypeStruct((NC, N, C_HID // 2), jnp.float32),
    )(x, W1, b1r, dis)


def _tc2_body(s_ref, dis_ref, w_ref, b_ref, g_ref):
    dis = dis_ref[...]
    s = jnp.concatenate([s_ref[0], s_ref[1]], axis=1)
    u = jnp.maximum(s * dis, 0.0)
    h = lax.dot_general(u, w_ref[...], (((1,), (1,)), ((), ())),
                        preferred_element_type=jnp.float32) + b_ref[...]
    g_ref[...] = h * dis


def _tc2(s1, dis, W2, b2r):
    return pl.pallas_call(
        _tc2_body,
        grid=(N // _BM,),
        in_specs=[
            pl.BlockSpec((NC, _BM, C_HID // 2), lambda i: (0, i, 0)),
            pl.BlockSpec((_BM, 1), lambda i: (i, 0)),
            pl.BlockSpec((C_OUT, C_HID), lambda i: (0, 0)),
            pl.BlockSpec((1, C_OUT), lambda i: (0, 0)),
        ],
        out_specs=pl.BlockSpec((_BM, C_OUT), lambda i: (i, 0)),
        out_shape=jax.ShapeDtypeStruct((N, C_OUT), jnp.float32),
    )(s1, dis, W2, b2r)


def _tc3_body(p_ref, g_ref, dis_ref, o_ref):
    o_ref[...] = (p_ref[0] + p_ref[1] - g_ref[...]) * dis_ref[...]


def _tc3(p2, g2, dis):
    return pl.pallas_call(
        _tc3_body,
        grid=(N // _BM,),
        in_specs=[
            pl.BlockSpec((NC, _BM, C_OUT), lambda i: (0, i, 0)),
            pl.BlockSpec((_BM, C_OUT), lambda i: (i, 0)),
            pl.BlockSpec((_BM, 1), lambda i: (i, 0)),
        ],
        out_specs=pl.BlockSpec((_BM, C_OUT), lambda i: (i, 0)),
        out_shape=jax.ShapeDtypeStruct((N, C_OUT), jnp.float32),
    )(p2, g2, dis)


def kernel(x, edge_index, W1, b1, W2, b2):
    ei = edge_index.astype(jnp.int32)
    row, col = ei[0], ei[1]
    idx1 = _chunk_desc_l1(row, col)
    idx2 = _chunk_desc_l2(row, col)
    degp = _deg_kernel(col).reshape(NW, N)
    dis = _tcdis(degp)
    g1s = _tc1(x, W1, b1.reshape(1, -1), dis)
    s1 = _scatter_l1(g1s.reshape(NC * N, C_HID // 2), idx1)
    g2 = _tc2(s1, dis, W2, b2.reshape(1, -1))
    p2 = _scatter_l2(g2, idx2)
    return _tc3(p2, g2, dis)


# speedup vs baseline: 1.5608x; 1.5608x over previous
import functools

import jax
import jax.numpy as jnp
from jax import lax
from jax.experimental import pallas as pl
from jax.experimental.pallas import tpu as pltpu
from jax.experimental.pallas import tpu_sc as plsc

N = 10000
E = 320000
C_IN = 128
C_HID = 256
C_OUT = 128

NC = 2
NT = 16
NW = NC * NT

_MESH = plsc.VectorSubcoreMesh(core_axis_name="c", subcore_axis_name="s")


EPW = E // NW


@functools.partial(
    pl.kernel, mesh=_MESH,
    out_type=jax.ShapeDtypeStruct((NW * N,), jnp.float32),
    scratch_types=[
        pltpu.VMEM((EPW,), jnp.int32),
        pltpu.VMEM((N,), jnp.float32),
    ],
    compiler_params=pltpu.CompilerParams(needs_layout_passes=False),
)
def _deg_kernel(col_hbm, out_hbm, colv, degv):
    wid = lax.axis_index("s") * NC + lax.axis_index("c")
    pltpu.sync_copy(col_hbm.at[pl.ds(wid * EPW, EPW)], colv)
    zeros = jnp.zeros((16,), jnp.float32)

    def _zero(i, carry):
        degv[pl.ds(i * 16, 16)] = zeros
        return carry

    lax.fori_loop(0, N // 16, _zero, 0)
    ones = jnp.ones((16,), jnp.float32)

    def _count(i, carry):
        idx = colv[pl.ds(i * 16, 16)]
        plsc.addupdate_scatter(degv, [idx], ones)
        return carry

    lax.fori_loop(0, EPW // 16, _count, 0)
    pltpu.sync_copy(degv, out_hbm.at[pl.ds(wid * N, N)])


def _make_scatter(H, K, NBUF, edge_split):
    EPT = (E // NC // NT) if edge_split else (E // NT)
    NCH = EPT // K
    NG = NCH // NBUF
    REM = NCH % NBUF
    RPT = 624
    TAIL0 = NT * RPT
    TAILN = N - TAIL0
    assert K % 8 == 0 and K <= 128 and NG >= 2

    @functools.partial(
        pl.kernel, mesh=_MESH,
        out_type=jax.ShapeDtypeStruct((NC, N, H), jnp.float32),
        scratch_types=[
            pltpu.VMEM((NBUF, K), jnp.int32),
            pltpu.VMEM((NBUF, K), jnp.int32),
            pltpu.VMEM((NBUF, K, H), jnp.float32),
            pltpu.VMEM_SHARED((N, H), jnp.float32),
        ] + [pltpu.SemaphoreType.DMA] * (3 * NBUF),
        compiler_params=pltpu.CompilerParams(needs_layout_passes=False),
    )
    def _scatter(g_hbm, row_hbm, col_hbm, out_hbm,
                 rowv, colv, rows, acc, *sems):
        isem = sems[:NBUF]
        gsem = sems[NBUF:2 * NBUF]
        ssem = sems[2 * NBUF:]
        sc = lax.axis_index("c")
        tid = lax.axis_index("s")
        r0 = tid * RPT
        if edge_split:
            gb = 0
            rb = sc * (E // NC) + tid * EPT
            cb = rb
        else:
            gb = sc * N
            rb = sc * E + tid * EPT
            cb = tid * EPT

        def _i_start(c, b):
            pltpu.async_copy(row_hbm.at[pl.ds(rb + c * K, K)],
                             rowv.at[b], isem[b])
            pltpu.async_copy(col_hbm.at[pl.ds(cb + c * K, K)],
                             colv.at[b], isem[b])

        def _i_wait(c, b):
            pltpu.make_async_copy(row_hbm.at[pl.ds(rb + c * K, K)],
                                  rowv.at[b], isem[b]).wait()
            pltpu.make_async_copy(col_hbm.at[pl.ds(cb + c * K, K)],
                                  colv.at[b], isem[b]).wait()

        def _g_start(b):
            pltpu.async_copy(g_hbm.at[rowv.at[b]], rows.at[b], gsem[b])

        def _g_wait(b):
            pltpu.make_async_copy(g_hbm.at[rowv.at[b]], rows.at[b],
                                  gsem[b]).wait()

        def _s_start(b):
            pltpu.async_copy(rows.at[b], acc.at[colv.at[b]], ssem[b],
                             add=True)

        def _s_wait(b):
            pltpu.make_async_copy(rows.at[b], acc.at[colv.at[b]],
                                  ssem[b]).wait()

        pltpu.sync_copy(g_hbm.at[pl.ds(gb + r0, RPT)], acc.at[pl.ds(r0, RPT)])

        @pl.when(tid == NT - 1)
        def _init_tail():
            pltpu.sync_copy(g_hbm.at[pl.ds(gb + TAIL0, TAILN)],
                            acc.at[pl.ds(TAIL0, TAILN)])

        plsc.subcore_barrier()
        _i_start(0, 0)

        def _group(p, carry):
            for b in range(NBUF):
                c = p * NBUF + b
                b1 = (b + 1) % NBUF
                bp = (b - 1) % NBUF
                _i_wait(c, b)
                _g_start(b)

                @pl.when(c >= NBUF - 1)
                def _free_next():
                    _s_wait(b1)

                @pl.when(c + 1 < NCH)
                def _prefetch():
                    _i_start(c + 1, b1)

                @pl.when(c >= 1)
                def _drain_prev():
                    _g_wait(bp)
                    _s_start(bp)
            return carry

        lax.fori_loop(0, NG, _group, 0)
        for t in range(REM):
            c = NG * NBUF + t
            b = t
            if t > 0:
                _s_wait(b)
                _i_start(c, b)
            _i_wait(c, b)
            _g_start(b)
            bp = (b - 1) % NBUF
            _g_wait(bp)
            _s_start(bp)
        lastb = (NCH - 1) % NBUF
        _g_wait(lastb)
        _s_start(lastb)
        for b in range(0 if REM else 1, NBUF):
            _s_wait(b)
        plsc.subcore_barrier()
        pltpu.sync_copy(acc.at[pl.ds(r0, RPT)], out_hbm.at[sc, pl.ds(r0, RPT)])

        @pl.when(tid == NT - 1)
        def _out_tail():
            pltpu.sync_copy(acc.at[pl.ds(TAIL0, TAILN)],
                            out_hbm.at[sc, pl.ds(TAIL0, TAILN)])

    return _scatter


_scatter_l1 = _make_scatter(C_HID // 2, 80, 4, False)
_scatter_l2 = _make_scatter(C_OUT, 80, 4, True)


_BM = 1000


def _tcdis_body(degp_ref, dis_ref):
    deg = jnp.sum(degp_ref[...], axis=0) + 1.0
    dis_ref[...] = lax.rsqrt(deg).reshape(-1, 1)


def _tcdis(degp):
    return pl.pallas_call(
        _tcdis_body,
        grid=(1,),
        in_specs=[pl.BlockSpec((NW, N), lambda i: (0, 0))],
        out_specs=pl.BlockSpec((N, 1), lambda i: (0, 0)),
        out_shape=jax.ShapeDtypeStruct((N, 1), jnp.float32),
    )(degp)


def _tc1_body(x_ref, w_ref, b_ref, dis_ref, g_ref):
    dis = dis_ref[...]
    h = lax.dot_general(x_ref[...], w_ref[...], (((1,), (1,)), ((), ())),
                        preferred_element_type=jnp.float32) + b_ref[...]
    g = h * dis
    g_ref[0] = g[:, :C_HID // 2]
    g_ref[1] = g[:, C_HID // 2:]


def _tc1(x, W1, b1r, dis):
    return pl.pallas_call(
        _tc1_body,
        grid=(N // _BM,),
        in_specs=[
            pl.BlockSpec((_BM, C_IN), lambda i: (i, 0)),
            pl.BlockSpec((C_HID, C_IN), lambda i: (0, 0)),
            pl.BlockSpec((1, C_HID), lambda i: (0, 0)),
            pl.BlockSpec((_BM, 1), lambda i: (i, 0)),
        ],
        out_specs=pl.BlockSpec((NC, _BM, C_HID // 2), lambda i: (0, i, 0)),
        out_shape=jax.ShapeDtypeStruct((NC, N, C_HID // 2), jnp.float32),
    )(x, W1, b1r, dis)


def _tc2_body(s_ref, dis_ref, w_ref, b_ref, g_ref):
    dis = dis_ref[...]
    s = jnp.concatenate([s_ref[0], s_ref[1]], axis=1)
    u = jnp.maximum(s * dis, 0.0)
    h = lax.dot_general(u, w_ref[...], (((1,), (1,)), ((), ())),
                        preferred_element_type=jnp.float32) + b_ref[...]
    g_ref[...] = h * dis


def _tc2(s1, dis, W2, b2r):
    return pl.pallas_call(
        _tc2_body,
        grid=(N // _BM,),
        in_specs=[
            pl.BlockSpec((NC, _BM, C_HID // 2), lambda i: (0, i, 0)),
            pl.BlockSpec((_BM, 1), lambda i: (i, 0)),
            pl.BlockSpec((C_OUT, C_HID), lambda i: (0, 0)),
            pl.BlockSpec((1, C_OUT), lambda i: (0, 0)),
        ],
        out_specs=pl.BlockSpec((_BM, C_OUT), lambda i: (i, 0)),
        out_shape=jax.ShapeDtypeStruct((N, C_OUT), jnp.float32),
    )(s1, dis, W2, b2r)


def _tc3_body(p_ref, g_ref, dis_ref, o_ref):
    o_ref[...] = (p_ref[0] + p_ref[1] - g_ref[...]) * dis_ref[...]


def _tc3(p2, g2, dis):
    return pl.pallas_call(
        _tc3_body,
        grid=(N // _BM,),
        in_specs=[
            pl.BlockSpec((NC, _BM, C_OUT), lambda i: (0, i, 0)),
            pl.BlockSpec((_BM, C_OUT), lambda i: (i, 0)),
            pl.BlockSpec((_BM, 1), lambda i: (i, 0)),
        ],
        out_specs=pl.BlockSpec((_BM, C_OUT), lambda i: (i, 0)),
        out_shape=jax.ShapeDtypeStruct((N, C_OUT), jnp.float32),
    )(p2, g2, dis)


def kernel(x, edge_index, W1, b1, W2, b2):
    ei = edge_index.astype(jnp.int32)
    row, col = ei[0], ei[1]
    rowcat = jnp.concatenate([row, row + N])
    degp = _deg_kernel(col).reshape(NW, N)
    dis = _tcdis(degp)
    g1s = _tc1(x, W1, b1.reshape(1, -1), dis)
    s1 = _scatter_l1(g1s.reshape(NC * N, C_HID // 2), rowcat, col)
    g2 = _tc2(s1, dis, W2, b2.reshape(1, -1))
    p2 = _scatter_l2(g2, row, col)
    return _tc3(p2, g2, dis)


# speedup vs baseline: 1.6221x; 1.0393x over previous
import functools

import jax
import jax.numpy as jnp
from jax import lax
from jax.experimental import pallas as pl
from jax.experimental.pallas import tpu as pltpu
from jax.experimental.pallas import tpu_sc as plsc

N = 10000
E = 320000
C_IN = 128
C_HID = 256
C_OUT = 128

NC = 2
NT = 16
NW = NC * NT

_MESH = plsc.VectorSubcoreMesh(core_axis_name="c", subcore_axis_name="s")


EPW = E // NW


@functools.partial(
    pl.kernel, mesh=_MESH,
    out_type=jax.ShapeDtypeStruct((NW * N,), jnp.float32),
    scratch_types=[
        pltpu.VMEM((EPW,), jnp.int32),
        pltpu.VMEM((N,), jnp.float32),
    ],
    compiler_params=pltpu.CompilerParams(needs_layout_passes=False),
)
def _deg_kernel(ei_hbm, out_hbm, colv, degv):
    wid = lax.axis_index("s") * NC + lax.axis_index("c")
    pltpu.sync_copy(ei_hbm.at[pl.ds(E + wid * EPW, EPW)], colv)
    zeros = jnp.zeros((16,), jnp.float32)

    def _zero(i, carry):
        degv[pl.ds(i * 16, 16)] = zeros
        return carry

    lax.fori_loop(0, N // 16, _zero, 0)
    ones = jnp.ones((16,), jnp.float32)

    def _count(i, carry):
        idx = colv[pl.ds(i * 16, 16)]
        plsc.addupdate_scatter(degv, [idx], ones)
        return carry

    lax.fori_loop(0, EPW // 16, _count, 0)
    pltpu.sync_copy(degv, out_hbm.at[pl.ds(wid * N, N)])


def _make_scatter(H, K, NBUF, edge_split):
    EPT = (E // NC // NT) if edge_split else (E // NT)
    NCH = EPT // K
    NG = NCH // NBUF
    REM = NCH % NBUF
    RPT = 624
    TAIL0 = NT * RPT
    TAILN = N - TAIL0
    assert K % 8 == 0 and K <= 128 and NG >= 2

    _SCRATCH = [
        pltpu.VMEM((NBUF, K), jnp.int32),
        pltpu.VMEM((NBUF, K), jnp.int32),
        pltpu.VMEM((NBUF, K, H), jnp.float32),
        pltpu.VMEM_SHARED((N, H), jnp.float32),
    ] + [pltpu.SemaphoreType.DMA] * (3 * NBUF)

    def _scatter_body(tabs, ei_hbm, out_hbm, rowv, colv, rows, acc, sems):
        isem = sems[:NBUF]
        gsem = sems[NBUF:2 * NBUF]
        ssem = sems[2 * NBUF:]
        sc = lax.axis_index("c")
        tid = lax.axis_index("s")
        r0 = tid * RPT
        if edge_split:
            rb = sc * (E // NC) + tid * EPT
        else:
            rb = tid * EPT
        cb = E + rb

        def _i_start(c, b):
            pltpu.async_copy(ei_hbm.at[pl.ds(rb + c * K, K)],
                             rowv.at[b], isem[b])
            pltpu.async_copy(ei_hbm.at[pl.ds(cb + c * K, K)],
                             colv.at[b], isem[b])

        def _i_wait(c, b):
            pltpu.make_async_copy(ei_hbm.at[pl.ds(rb + c * K, K)],
                                  rowv.at[b], isem[b]).wait()
            pltpu.make_async_copy(ei_hbm.at[pl.ds(cb + c * K, K)],
                                  colv.at[b], isem[b]).wait()

        def _per_table(fn):
            if len(tabs) == 1:
                fn(tabs[0])
            else:
                @pl.when(sc == 0)
                def _t0():
                    fn(tabs[0])

                @pl.when(sc == 1)
                def _t1():
                    fn(tabs[1])

        def _g_start(b):
            _per_table(lambda t: pltpu.async_copy(
                t.at[rowv.at[b]], rows.at[b], gsem[b]))

        def _g_wait(b):
            _per_table(lambda t: pltpu.make_async_copy(
                t.at[rowv.at[b]], rows.at[b], gsem[b]).wait())

        def _s_start(b):
            pltpu.async_copy(rows.at[b], acc.at[colv.at[b]], ssem[b],
                             add=True)

        def _s_wait(b):
            pltpu.make_async_copy(rows.at[b], acc.at[colv.at[b]],
                                  ssem[b]).wait()

        _per_table(lambda t: pltpu.sync_copy(t.at[pl.ds(r0, RPT)],
                                             acc.at[pl.ds(r0, RPT)]))

        @pl.when(tid == NT - 1)
        def _init_tail():
            _per_table(lambda t: pltpu.sync_copy(t.at[pl.ds(TAIL0, TAILN)],
                                                 acc.at[pl.ds(TAIL0, TAILN)]))

        plsc.subcore_barrier()
        _i_start(0, 0)

        def _group(p, carry):
            for b in range(NBUF):
                c = p * NBUF + b
                b1 = (b + 1) % NBUF
                bp = (b - 1) % NBUF
                _i_wait(c, b)
                _g_start(b)

                @pl.when(c >= NBUF - 1)
                def _free_next():
                    _s_wait(b1)

                @pl.when(c + 1 < NCH)
                def _prefetch():
                    _i_start(c + 1, b1)

                @pl.when(c >= 1)
                def _drain_prev():
                    _g_wait(bp)
                    _s_start(bp)
            return carry

        lax.fori_loop(0, NG, _group, 0)
        for t in range(REM):
            c = NG * NBUF + t
            b = t
            if t > 0:
                _s_wait(b)
                _i_start(c, b)
            _i_wait(c, b)
            _g_start(b)
            bp = (b - 1) % NBUF
            _g_wait(bp)
            _s_start(bp)
        lastb = (NCH - 1) % NBUF
        _g_wait(lastb)
        _s_start(lastb)
        for b in range(0 if REM else 1, NBUF):
            _s_wait(b)
        plsc.subcore_barrier()
        pltpu.sync_copy(acc.at[pl.ds(r0, RPT)], out_hbm.at[sc, pl.ds(r0, RPT)])

        @pl.when(tid == NT - 1)
        def _out_tail():
            pltpu.sync_copy(acc.at[pl.ds(TAIL0, TAILN)],
                            out_hbm.at[sc, pl.ds(TAIL0, TAILN)])

    if edge_split:
        @functools.partial(
            pl.kernel, mesh=_MESH,
            out_type=jax.ShapeDtypeStruct((NC, N, H), jnp.float32),
            scratch_types=_SCRATCH,
            compiler_params=pltpu.CompilerParams(needs_layout_passes=False),
        )
        def _scatter(g_hbm, ei_hbm, out_hbm, rowv, colv, rows, acc, *sems):
            _scatter_body((g_hbm,), ei_hbm, out_hbm, rowv, colv, rows, acc,
                          sems)
    else:
        @functools.partial(
            pl.kernel, mesh=_MESH,
            out_type=jax.ShapeDtypeStruct((NC, N, H), jnp.float32),
            scratch_types=_SCRATCH,
            compiler_params=pltpu.CompilerParams(needs_layout_passes=False),
        )
        def _scatter(glo_hbm, ghi_hbm, ei_hbm, out_hbm,
                     rowv, colv, rows, acc, *sems):
            _scatter_body((glo_hbm, ghi_hbm), ei_hbm, out_hbm,
                          rowv, colv, rows, acc, sems)

    return _scatter


_scatter_l1 = _make_scatter(C_HID // 2, 80, 4, False)
_scatter_l2 = _make_scatter(C_OUT, 80, 4, True)


_BM = 1000


def _tcdis_body(degp_ref, dis_ref):
    deg = degp_ref[pl.ds(0, N)] + 1.0
    for w in range(1, NW):
        deg = deg + degp_ref[pl.ds(w * N, N)]
    dis_ref[...] = lax.rsqrt(deg).reshape(-1, 1)


def _tcdis(degp_flat):
    return pl.pallas_call(
        _tcdis_body,
        grid=(1,),
        in_specs=[pl.BlockSpec((NW * N,), lambda i: (0,))],
        out_specs=pl.BlockSpec((N, 1), lambda i: (0, 0)),
        out_shape=jax.ShapeDtypeStruct((N, 1), jnp.float32),
    )(degp_flat)


def _tc1_body(x_ref, w_ref, b_ref, dis_ref, glo_ref, ghi_ref):
    dis = dis_ref[...]
    h = lax.dot_general(x_ref[...], w_ref[...], (((1,), (1,)), ((), ())),
                        preferred_element_type=jnp.float32) + b_ref[...]
    g = h * dis
    glo_ref[...] = g[:, :C_HID // 2]
    ghi_ref[...] = g[:, C_HID // 2:]


def _tc1(x, W1, b1r, dis):
    return pl.pallas_call(
        _tc1_body,
        grid=(N // _BM,),
        in_specs=[
            pl.BlockSpec((_BM, C_IN), lambda i: (i, 0)),
            pl.BlockSpec((C_HID, C_IN), lambda i: (0, 0)),
            pl.BlockSpec((1, C_HID), lambda i: (0, 0)),
            pl.BlockSpec((_BM, 1), lambda i: (i, 0)),
        ],
        out_specs=[
            pl.BlockSpec((_BM, C_HID // 2), lambda i: (i, 0)),
            pl.BlockSpec((_BM, C_HID // 2), lambda i: (i, 0)),
        ],
        out_shape=[
            jax.ShapeDtypeStruct((N, C_HID // 2), jnp.float32),
            jax.ShapeDtypeStruct((N, C_HID // 2), jnp.float32),
        ],
    )(x, W1, b1r, dis)


def _tc2_body(s_ref, dis_ref, w_ref, b_ref, g_ref):
    dis = dis_ref[...]
    s = jnp.concatenate([s_ref[0], s_ref[1]], axis=1)
    u = jnp.maximum(s * dis, 0.0)
    h = lax.dot_general(u, w_ref[...], (((1,), (1,)), ((), ())),
                        preferred_element_type=jnp.float32) + b_ref[...]
    g_ref[...] = h * dis


def _tc2(s1, dis, W2, b2r):
    return pl.pallas_call(
        _tc2_body,
        grid=(N // _BM,),
        in_specs=[
            pl.BlockSpec((NC, _BM, C_HID // 2), lambda i: (0, i, 0)),
            pl.BlockSpec((_BM, 1), lambda i: (i, 0)),
            pl.BlockSpec((C_OUT, C_HID), lambda i: (0, 0)),
            pl.BlockSpec((1, C_OUT), lambda i: (0, 0)),
        ],
        out_specs=pl.BlockSpec((_BM, C_OUT), lambda i: (i, 0)),
        out_shape=jax.ShapeDtypeStruct((N, C_OUT), jnp.float32),
    )(s1, dis, W2, b2r)


def _tc3_body(p_ref, g_ref, dis_ref, o_ref):
    o_ref[...] = (p_ref[0] + p_ref[1] - g_ref[...]) * dis_ref[...]


def _tc3(p2, g2, dis):
    return pl.pallas_call(
        _tc3_body,
        grid=(N // _BM,),
        in_specs=[
            pl.BlockSpec((NC, _BM, C_OUT), lambda i: (0, i, 0)),
            pl.BlockSpec((_BM, C_OUT), lambda i: (i, 0)),
            pl.BlockSpec((_BM, 1), lambda i: (i, 0)),
        ],
        out_specs=pl.BlockSpec((_BM, C_OUT), lambda i: (i, 0)),
        out_shape=jax.ShapeDtypeStruct((N, C_OUT), jnp.float32),
    )(p2, g2, dis)


def kernel(x, edge_index, W1, b1, W2, b2):
    eif = edge_index.astype(jnp.int32).reshape(2 * E)
    degp = _deg_kernel(eif)
    dis = _tcdis(degp)
    glo, ghi = _tc1(x, W1, b1.reshape(1, -1), dis)
    s1 = _scatter_l1(glo, ghi, eif)
    g2 = _tc2(s1, dis, W2, b2.reshape(1, -1))
    p2 = _scatter_l2(g2, eif)
    return _tc3(p2, g2, dis)


# speedup vs baseline: 1.6506x; 1.0176x over previous
import functools

import jax
import jax.numpy as jnp
from jax import lax
from jax.experimental import pallas as pl
from jax.experimental.pallas import tpu as pltpu
from jax.experimental.pallas import tpu_sc as plsc

N = 10000
E = 320000
C_IN = 128
C_HID = 256
C_OUT = 128

NC = 2
NT = 16
NW = NC * NT

_MESH = plsc.VectorSubcoreMesh(core_axis_name="c", subcore_axis_name="s")


EPW = E // NW


@functools.partial(
    pl.kernel, mesh=_MESH,
    out_type=jax.ShapeDtypeStruct((NW * N,), jnp.float32),
    scratch_types=[
        pltpu.VMEM((EPW,), jnp.int32),
        pltpu.VMEM((N,), jnp.float32),
    ],
    compiler_params=pltpu.CompilerParams(needs_layout_passes=False),
)
def _deg_kernel(ei_hbm, out_hbm, colv, degv):
    wid = lax.axis_index("s") * NC + lax.axis_index("c")
    pltpu.sync_copy(ei_hbm.at[pl.ds(E + wid * EPW, EPW)], colv)
    zeros = jnp.zeros((16,), jnp.float32)

    def _zero(i, carry):
        degv[pl.ds(i * 16, 16)] = zeros
        return carry

    lax.fori_loop(0, N // 16, _zero, 0)
    ones = jnp.ones((16,), jnp.float32)

    def _count(i, carry):
        idx = colv[pl.ds(i * 16, 16)]
        plsc.addupdate_scatter(degv, [idx], ones)
        return carry

    lax.fori_loop(0, EPW // 16, _count, 0)
    pltpu.sync_copy(degv, out_hbm.at[pl.ds(wid * N, N)])


def _make_scatter(H, K, NBUF, edge_split):
    EPT = (E // NC // NT) if edge_split else (E // NT)
    NCH = EPT // K
    NG = NCH // NBUF
    REM = NCH % NBUF
    RPT = 624
    TAIL0 = NT * RPT
    TAILN = N - TAIL0
    assert K % 8 == 0 and K <= 128 and NG >= 2

    _SCRATCH = [
        pltpu.VMEM((NBUF, K), jnp.int32),
        pltpu.VMEM((NBUF, K), jnp.int32),
        pltpu.VMEM((NBUF, K, H), jnp.float32),
        pltpu.VMEM_SHARED((N, H), jnp.float32),
    ] + [pltpu.SemaphoreType.DMA] * (3 * NBUF)

    def _scatter_body(tabs, ei_hbm, out_hbm, rowv, colv, rows, acc, sems):
        isem = sems[:NBUF]
        gsem = sems[NBUF:2 * NBUF]
        ssem = sems[2 * NBUF:]
        sc = lax.axis_index("c")
        tid = lax.axis_index("s")
        r0 = tid * RPT
        if edge_split:
            rb = sc * (E // NC) + tid * EPT
        else:
            rb = tid * EPT
        cb = E + rb

        def _i_start(c, b):
            pltpu.async_copy(ei_hbm.at[pl.ds(rb + c * K, K)],
                             rowv.at[b], isem[b])
            pltpu.async_copy(ei_hbm.at[pl.ds(cb + c * K, K)],
                             colv.at[b], isem[b])

        def _i_wait(c, b):
            pltpu.make_async_copy(ei_hbm.at[pl.ds(rb + c * K, K)],
                                  rowv.at[b], isem[b]).wait()
            pltpu.make_async_copy(ei_hbm.at[pl.ds(cb + c * K, K)],
                                  colv.at[b], isem[b]).wait()

        def _per_table(fn):
            if len(tabs) == 1:
                fn(tabs[0])
            else:
                @pl.when(sc == 0)
                def _t0():
                    fn(tabs[0])

                @pl.when(sc == 1)
                def _t1():
                    fn(tabs[1])

        def _g_start(b):
            _per_table(lambda t: pltpu.async_copy(
                t.at[rowv.at[b]], rows.at[b], gsem[b]))

        def _g_wait(b):
            _per_table(lambda t: pltpu.make_async_copy(
                t.at[rowv.at[b]], rows.at[b], gsem[b]).wait())

        def _s_start(b):
            pltpu.async_copy(rows.at[b], acc.at[colv.at[b]], ssem[b],
                             add=True)

        def _s_wait(b):
            pltpu.make_async_copy(rows.at[b], acc.at[colv.at[b]],
                                  ssem[b]).wait()

        _per_table(lambda t: pltpu.sync_copy(t.at[pl.ds(r0, RPT)],
                                             acc.at[pl.ds(r0, RPT)]))

        @pl.when(tid == NT - 1)
        def _init_tail():
            _per_table(lambda t: pltpu.sync_copy(t.at[pl.ds(TAIL0, TAILN)],
                                                 acc.at[pl.ds(TAIL0, TAILN)]))

        plsc.subcore_barrier()
        _i_start(0, 0)

        def _group(p, carry):
            for b in range(NBUF):
                c = p * NBUF + b
                b1 = (b + 1) % NBUF
                bp = (b - 1) % NBUF
                _i_wait(c, b)
                _g_start(b)

                @pl.when(c >= NBUF - 1)
                def _free_next():
                    _s_wait(b1)

                @pl.when(c + 1 < NCH)
                def _prefetch():
                    _i_start(c + 1, b1)

                @pl.when(c >= 1)
                def _drain_prev():
                    _g_wait(bp)
                    _s_start(bp)
            return carry

        lax.fori_loop(0, NG, _group, 0)
        for t in range(REM):
            c = NG * NBUF + t
            b = t
            if t > 0:
                _s_wait(b)
                _i_start(c, b)
            _i_wait(c, b)
            _g_start(b)
            bp = (b - 1) % NBUF
            _g_wait(bp)
            _s_start(bp)
        lastb = (NCH - 1) % NBUF
        _g_wait(lastb)
        _s_start(lastb)
        for b in range(0 if REM else 1, NBUF):
            _s_wait(b)
        plsc.subcore_barrier()
        pltpu.sync_copy(acc.at[pl.ds(r0, RPT)], out_hbm.at[sc, pl.ds(r0, RPT)])

        @pl.when(tid == NT - 1)
        def _out_tail():
            pltpu.sync_copy(acc.at[pl.ds(TAIL0, TAILN)],
                            out_hbm.at[sc, pl.ds(TAIL0, TAILN)])

    if edge_split:
        @functools.partial(
            pl.kernel, mesh=_MESH,
            out_type=jax.ShapeDtypeStruct((NC, N, H), jnp.float32),
            scratch_types=_SCRATCH,
            compiler_params=pltpu.CompilerParams(needs_layout_passes=False),
        )
        def _scatter(g_hbm, ei_hbm, out_hbm, rowv, colv, rows, acc, *sems):
            _scatter_body((g_hbm,), ei_hbm, out_hbm, rowv, colv, rows, acc,
                          sems)
    else:
        @functools.partial(
            pl.kernel, mesh=_MESH,
            out_type=jax.ShapeDtypeStruct((NC, N, H), jnp.float32),
            scratch_types=_SCRATCH,
            compiler_params=pltpu.CompilerParams(needs_layout_passes=False),
        )
        def _scatter(glo_hbm, ghi_hbm, ei_hbm, out_hbm,
                     rowv, colv, rows, acc, *sems):
            _scatter_body((glo_hbm, ghi_hbm), ei_hbm, out_hbm,
                          rowv, colv, rows, acc, sems)

    return _scatter


_scatter_l1 = _make_scatter(C_HID // 2, 80, 4, False)
_scatter_l2 = _make_scatter(C_OUT, 80, 4, True)


_BM = 2000


def _tcdis_body(degp_ref, dis_ref):
    deg = degp_ref[pl.ds(0, N)] + 1.0
    for w in range(1, NW):
        deg = deg + degp_ref[pl.ds(w * N, N)]
    dis_ref[...] = lax.rsqrt(deg).reshape(-1, 1)


def _tcdis(degp_flat):
    return pl.pallas_call(
        _tcdis_body,
        grid=(1,),
        in_specs=[pl.BlockSpec((NW * N,), lambda i: (0,))],
        out_specs=pl.BlockSpec((N, 1), lambda i: (0, 0)),
        out_shape=jax.ShapeDtypeStruct((N, 1), jnp.float32),
    )(degp_flat)


def _tc1_body(x_ref, w_ref, b_ref, dis_ref, glo_ref, ghi_ref):
    dis = dis_ref[...]
    h = lax.dot_general(x_ref[...], w_ref[...], (((1,), (1,)), ((), ())),
                        preferred_element_type=jnp.float32) + b_ref[...]
    g = h * dis
    glo_ref[...] = g[:, :C_HID // 2]
    ghi_ref[...] = g[:, C_HID // 2:]


def _tc1(x, W1, b1r, dis):
    return pl.pallas_call(
        _tc1_body,
        grid=(N // _BM,),
        in_specs=[
            pl.BlockSpec((_BM, C_IN), lambda i: (i, 0)),
            pl.BlockSpec((C_HID, C_IN), lambda i: (0, 0)),
            pl.BlockSpec((1, C_HID), lambda i: (0, 0)),
            pl.BlockSpec((_BM, 1), lambda i: (i, 0)),
        ],
        out_specs=[
            pl.BlockSpec((_BM, C_HID // 2), lambda i: (i, 0)),
            pl.BlockSpec((_BM, C_HID // 2), lambda i: (i, 0)),
        ],
        out_shape=[
            jax.ShapeDtypeStruct((N, C_HID // 2), jnp.float32),
            jax.ShapeDtypeStruct((N, C_HID // 2), jnp.float32),
        ],
    )(x, W1, b1r, dis)


def _tc2_body(s_ref, dis_ref, w_ref, b_ref, g_ref):
    dis = dis_ref[...]
    s = jnp.concatenate([s_ref[0], s_ref[1]], axis=1)
    u = jnp.maximum(s * dis, 0.0)
    h = lax.dot_general(u, w_ref[...], (((1,), (1,)), ((), ())),
                        preferred_element_type=jnp.float32) + b_ref[...]
    g_ref[...] = h * dis


def _tc2(s1, dis, W2, b2r):
    return pl.pallas_call(
        _tc2_body,
        grid=(N // _BM,),
        in_specs=[
            pl.BlockSpec((NC, _BM, C_HID // 2), lambda i: (0, i, 0)),
            pl.BlockSpec((_BM, 1), lambda i: (i, 0)),
            pl.BlockSpec((C_OUT, C_HID), lambda i: (0, 0)),
            pl.BlockSpec((1, C_OUT), lambda i: (0, 0)),
        ],
        out_specs=pl.BlockSpec((_BM, C_OUT), lambda i: (i, 0)),
        out_shape=jax.ShapeDtypeStruct((N, C_OUT), jnp.float32),
    )(s1, dis, W2, b2r)


def _tc3_body(p_ref, g_ref, dis_ref, o_ref):
    o_ref[...] = (p_ref[0] + p_ref[1] - g_ref[...]) * dis_ref[...]


def _tc3(p2, g2, dis):
    return pl.pallas_call(
        _tc3_body,
        grid=(N // _BM,),
        in_specs=[
            pl.BlockSpec((NC, _BM, C_OUT), lambda i: (0, i, 0)),
            pl.BlockSpec((_BM, C_OUT), lambda i: (i, 0)),
            pl.BlockSpec((_BM, 1), lambda i: (i, 0)),
        ],
        out_specs=pl.BlockSpec((_BM, C_OUT), lambda i: (i, 0)),
        out_shape=jax.ShapeDtypeStruct((N, C_OUT), jnp.float32),
    )(p2, g2, dis)


def kernel(x, edge_index, W1, b1, W2, b2):
    eif = edge_index.astype(jnp.int32).reshape(2 * E)
    degp = _deg_kernel(eif)
    dis = _tcdis(degp)
    glo, ghi = _tc1(x, W1, b1.reshape(1, -1), dis)
    s1 = _scatter_l1(glo, ghi, eif)
    g2 = _tc2(s1, dis, W2, b2.reshape(1, -1))
    p2 = _scatter_l2(g2, eif)
    return _tc3(p2, g2, dis)


# speedup vs baseline: 1.6782x; 1.0167x over previous
import functools

import jax
import jax.numpy as jnp
from jax import lax
from jax.experimental import pallas as pl
from jax.experimental.pallas import tpu as pltpu
from jax.experimental.pallas import tpu_sc as plsc

N = 10000
E = 320000
C_IN = 128
C_HID = 256
C_OUT = 128

NC = 2
NT = 16
NW = NC * NT

_MESH = plsc.VectorSubcoreMesh(core_axis_name="c", subcore_axis_name="s")


EPW = E // NW


@functools.partial(
    pl.kernel, mesh=_MESH,
    out_type=jax.ShapeDtypeStruct((NW * N,), jnp.float32),
    scratch_types=[
        pltpu.VMEM((EPW,), jnp.int32),
        pltpu.VMEM((N,), jnp.float32),
    ],
    compiler_params=pltpu.CompilerParams(needs_layout_passes=False),
)
def _deg_kernel(ei_hbm, out_hbm, colv, degv):
    wid = lax.axis_index("s") * NC + lax.axis_index("c")
    pltpu.sync_copy(ei_hbm.at[pl.ds(E + wid * EPW, EPW)], colv)
    zeros = jnp.zeros((16,), jnp.float32)

    def _zero(i, carry):
        degv[pl.ds(i * 16, 16)] = zeros
        return carry

    lax.fori_loop(0, N // 16, _zero, 0)
    ones = jnp.ones((16,), jnp.float32)

    def _count(i, carry):
        idx = colv[pl.ds(i * 16, 16)]
        plsc.addupdate_scatter(degv, [idx], ones)
        return carry

    lax.fori_loop(0, EPW // 16, _count, 0)
    pltpu.sync_copy(degv, out_hbm.at[pl.ds(wid * N, N)])


def _make_scatter(H, K, NBUF, edge_split):
    EPT = (E // NC // NT) if edge_split else (E // NT)
    NCH = EPT // K
    NG = NCH // NBUF
    REM = NCH % NBUF
    RPT = 624
    TAIL0 = NT * RPT
    TAILN = N - TAIL0
    assert K % 8 == 0 and K <= 128 and NG >= 2

    _SCRATCH = [
        pltpu.VMEM((NBUF, K), jnp.int32),
        pltpu.VMEM((NBUF, K), jnp.int32),
        pltpu.VMEM((NBUF, K, H), jnp.float32),
        pltpu.VMEM_SHARED((N, H), jnp.float32),
    ] + [pltpu.SemaphoreType.DMA] * (3 * NBUF)

    def _scatter_body(tabs, ei_hbm, out_hbm, rowv, colv, rows, acc, sems):
        isem = sems[:NBUF]
        gsem = sems[NBUF:2 * NBUF]
        ssem = sems[2 * NBUF:]
        sc = lax.axis_index("c")
        tid = lax.axis_index("s")
        r0 = tid * RPT
        if edge_split:
            rb = sc * (E // NC) + tid * EPT
        else:
            rb = tid * EPT
        cb = E + rb

        def _i_start(c, b):
            pltpu.async_copy(ei_hbm.at[pl.ds(rb + c * K, K)],
                             rowv.at[b], isem[b])
            pltpu.async_copy(ei_hbm.at[pl.ds(cb + c * K, K)],
                             colv.at[b], isem[b])

        def _i_wait(c, b):
            pltpu.make_async_copy(ei_hbm.at[pl.ds(rb + c * K, K)],
                                  rowv.at[b], isem[b]).wait()
            pltpu.make_async_copy(ei_hbm.at[pl.ds(cb + c * K, K)],
                                  colv.at[b], isem[b]).wait()

        def _per_table(fn):
            if len(tabs) == 1:
                fn(tabs[0])
            else:
                @pl.when(sc == 0)
                def _t0():
                    fn(tabs[0])

                @pl.when(sc == 1)
                def _t1():
                    fn(tabs[1])

        def _g_start(b):
            _per_table(lambda t: pltpu.async_copy(
                t.at[rowv.at[b]], rows.at[b], gsem[b]))

        def _g_wait(b):
            _per_table(lambda t: pltpu.make_async_copy(
                t.at[rowv.at[b]], rows.at[b], gsem[b]).wait())

        def _s_start(b):
            pltpu.async_copy(rows.at[b], acc.at[colv.at[b]], ssem[b],
                             add=True)

        def _s_wait(b):
            pltpu.make_async_copy(rows.at[b], acc.at[colv.at[b]],
                                  ssem[b]).wait()

        _per_table(lambda t: pltpu.sync_copy(t.at[pl.ds(r0, RPT)],
                                             acc.at[pl.ds(r0, RPT)]))

        @pl.when(tid == NT - 1)
        def _init_tail():
            _per_table(lambda t: pltpu.sync_copy(t.at[pl.ds(TAIL0, TAILN)],
                                                 acc.at[pl.ds(TAIL0, TAILN)]))

        plsc.subcore_barrier()
        _i_start(0, 0)

        def _group(p, carry):
            for b in range(NBUF):
                c = p * NBUF + b
                b1 = (b + 1) % NBUF
                bp = (b - 1) % NBUF
                _i_wait(c, b)
                _g_start(b)

                @pl.when(c >= NBUF - 1)
                def _free_next():
                    _s_wait(b1)

                @pl.when(c + 1 < NCH)
                def _prefetch():
                    _i_start(c + 1, b1)

                @pl.when(c >= 1)
                def _drain_prev():
                    _g_wait(bp)
                    _s_start(bp)
            return carry

        lax.fori_loop(0, NG, _group, 0)
        for t in range(REM):
            c = NG * NBUF + t
            b = t
            if t > 0:
                _s_wait(b)
                _i_start(c, b)
            _i_wait(c, b)
            _g_start(b)
            bp = (b - 1) % NBUF
            _g_wait(bp)
            _s_start(bp)
        lastb = (NCH - 1) % NBUF
        _g_wait(lastb)
        _s_start(lastb)
        for b in range(0 if REM else 1, NBUF):
            _s_wait(b)
        plsc.subcore_barrier()
        pltpu.sync_copy(acc.at[pl.ds(r0, RPT)], out_hbm.at[sc, pl.ds(r0, RPT)])

        @pl.when(tid == NT - 1)
        def _out_tail():
            pltpu.sync_copy(acc.at[pl.ds(TAIL0, TAILN)],
                            out_hbm.at[sc, pl.ds(TAIL0, TAILN)])

    if edge_split:
        @functools.partial(
            pl.kernel, mesh=_MESH,
            out_type=jax.ShapeDtypeStruct((NC, N, H), jnp.float32),
            scratch_types=_SCRATCH,
            compiler_params=pltpu.CompilerParams(needs_layout_passes=False),
        )
        def _scatter(g_hbm, ei_hbm, out_hbm, rowv, colv, rows, acc, *sems):
            _scatter_body((g_hbm,), ei_hbm, out_hbm, rowv, colv, rows, acc,
                          sems)
    else:
        @functools.partial(
            pl.kernel, mesh=_MESH,
            out_type=jax.ShapeDtypeStruct((NC, N, H), jnp.float32),
            scratch_types=_SCRATCH,
            compiler_params=pltpu.CompilerParams(needs_layout_passes=False),
        )
        def _scatter(glo_hbm, ghi_hbm, ei_hbm, out_hbm,
                     rowv, colv, rows, acc, *sems):
            _scatter_body((glo_hbm, ghi_hbm), ei_hbm, out_hbm,
                          rowv, colv, rows, acc, sems)

    return _scatter


_scatter_l1 = _make_scatter(C_HID // 2, 80, 4, False)
_scatter_l2 = _make_scatter(C_OUT, 80, 4, True)


_BM = 2000


def _tc1_body(degp_ref, x_ref, w_ref, b_ref, glo_ref, ghi_ref, dis_ref):
    deg = degp_ref[pl.ds(0, N)] + 1.0
    for w in range(1, NW):
        deg = deg + degp_ref[pl.ds(w * N, N)]
    dis = lax.rsqrt(deg).reshape(-1, 1)
    h = lax.dot_general(x_ref[...], w_ref[...], (((1,), (1,)), ((), ())),
                        preferred_element_type=jnp.float32) + b_ref[...]
    g = h * dis
    glo_ref[...] = g[:, :C_HID // 2]
    ghi_ref[...] = g[:, C_HID // 2:]
    dis_ref[...] = dis


def _tc1(degp, x, W1, b1r):
    return pl.pallas_call(
        _tc1_body,
        grid=(1,),
        in_specs=[
            pl.BlockSpec((NW * N,), lambda i: (0,)),
            pl.BlockSpec((N, C_IN), lambda i: (0, 0)),
            pl.BlockSpec((C_HID, C_IN), lambda i: (0, 0)),
            pl.BlockSpec((1, C_HID), lambda i: (0, 0)),
        ],
        out_specs=[
            pl.BlockSpec((N, C_HID // 2), lambda i: (0, 0)),
            pl.BlockSpec((N, C_HID // 2), lambda i: (0, 0)),
            pl.BlockSpec((N, 1), lambda i: (0, 0)),
        ],
        out_shape=[
            jax.ShapeDtypeStruct((N, C_HID // 2), jnp.float32),
            jax.ShapeDtypeStruct((N, C_HID // 2), jnp.float32),
            jax.ShapeDtypeStruct((N, 1), jnp.float32),
        ],
    )(degp, x, W1, b1r)


def _tc2_body(s_ref, dis_ref, w_ref, b_ref, g_ref):
    dis = dis_ref[...]
    s = jnp.concatenate([s_ref[0], s_ref[1]], axis=1)
    u = jnp.maximum(s * dis, 0.0)
    h = lax.dot_general(u, w_ref[...], (((1,), (1,)), ((), ())),
                        preferred_element_type=jnp.float32) + b_ref[...]
    g_ref[...] = h * dis


def _tc2(s1, dis, W2, b2r):
    return pl.pallas_call(
        _tc2_body,
        grid=(N // _BM,),
        in_specs=[
            pl.BlockSpec((NC, _BM, C_HID // 2), lambda i: (0, i, 0)),
            pl.BlockSpec((_BM, 1), lambda i: (i, 0)),
            pl.BlockSpec((C_OUT, C_HID), lambda i: (0, 0)),
            pl.BlockSpec((1, C_OUT), lambda i: (0, 0)),
        ],
        out_specs=pl.BlockSpec((_BM, C_OUT), lambda i: (i, 0)),
        out_shape=jax.ShapeDtypeStruct((N, C_OUT), jnp.float32),
    )(s1, dis, W2, b2r)


def _tc3_body(p_ref, g_ref, dis_ref, o_ref):
    o_ref[...] = (p_ref[0] + p_ref[1] - g_ref[...]) * dis_ref[...]


def _tc3(p2, g2, dis):
    return pl.pallas_call(
        _tc3_body,
        grid=(N // _BM,),
        in_specs=[
            pl.BlockSpec((NC, _BM, C_OUT), lambda i: (0, i, 0)),
            pl.BlockSpec((_BM, C_OUT), lambda i: (i, 0)),
            pl.BlockSpec((_BM, 1), lambda i: (i, 0)),
        ],
        out_specs=pl.BlockSpec((_BM, C_OUT), lambda i: (i, 0)),
        out_shape=jax.ShapeDtypeStruct((N, C_OUT), jnp.float32),
    )(p2, g2, dis)


def kernel(x, edge_index, W1, b1, W2, b2):
    eif = edge_index.astype(jnp.int32).reshape(2 * E)
    degp = _deg_kernel(eif)
    glo, ghi, dis = _tc1(degp, x, W1, b1.reshape(1, -1))
    s1 = _scatter_l1(glo, ghi, eif)
    g2 = _tc2(s1, dis, W2, b2.reshape(1, -1))
    p2 = _scatter_l2(g2, eif)
    return _tc3(p2, g2, dis)


# speedup vs baseline: 1.7195x; 1.0246x over previous
import functools

import jax
import jax.numpy as jnp
from jax import lax
from jax.experimental import pallas as pl
from jax.experimental.pallas import tpu as pltpu
from jax.experimental.pallas import tpu_sc as plsc

N = 10000
E = 320000
C_IN = 128
C_HID = 256
C_OUT = 128

NC = 2
NT = 16
NW = NC * NT

_MESH = plsc.VectorSubcoreMesh(core_axis_name="c", subcore_axis_name="s")


EPW = E // NW


@functools.partial(
    pl.kernel, mesh=_MESH,
    out_type=jax.ShapeDtypeStruct((NW * N,), jnp.float32),
    scratch_types=[
        pltpu.VMEM((EPW,), jnp.int32),
        pltpu.VMEM((N,), jnp.float32),
    ],
    compiler_params=pltpu.CompilerParams(needs_layout_passes=False),
)
def _deg_kernel(ei_hbm, out_hbm, colv, degv):
    wid = lax.axis_index("s") * NC + lax.axis_index("c")
    pltpu.sync_copy(ei_hbm.at[pl.ds(E + wid * EPW, EPW)], colv)
    zeros = jnp.zeros((16,), jnp.float32)

    def _zero(i, carry):
        degv[pl.ds(i * 16, 16)] = zeros
        return carry

    lax.fori_loop(0, N // 16, _zero, 0)
    ones = jnp.ones((16,), jnp.float32)

    def _count(i, carry):
        idx = colv[pl.ds(i * 16, 16)]
        plsc.addupdate_scatter(degv, [idx], ones)
        return carry

    lax.fori_loop(0, EPW // 16, _count, 0)
    pltpu.sync_copy(degv, out_hbm.at[pl.ds(wid * N, N)])


def _make_scatter(H, K, NBUF, edge_split):
    EPT = (E // NC // NT) if edge_split else (E // NT)
    NCH = EPT // K
    NG = NCH // NBUF
    REM = NCH % NBUF
    RPT = 624
    TAIL0 = NT * RPT
    TAILN = N - TAIL0
    assert K % 8 == 0 and K <= 128 and NG >= 2

    _SCRATCH = [
        pltpu.VMEM((NBUF, K), jnp.int32),
        pltpu.VMEM((NBUF, K), jnp.int32),
        pltpu.VMEM((NBUF, K, H), jnp.float32),
        pltpu.VMEM_SHARED((N, H), jnp.float32),
    ] + [pltpu.SemaphoreType.DMA] * (3 * NBUF)

    def _scatter_body(tabs, ei_hbm, out_hbm, rowv, colv, rows, acc, sems):
        isem = sems[:NBUF]
        gsem = sems[NBUF:2 * NBUF]
        ssem = sems[2 * NBUF:]
        sc = lax.axis_index("c")
        tid = lax.axis_index("s")
        r0 = tid * RPT
        if edge_split:
            rb = sc * (E // NC) + tid * EPT
        else:
            rb = tid * EPT
        cb = E + rb

        def _i_start(c, b):
            pltpu.async_copy(ei_hbm.at[pl.ds(rb + c * K, K)],
                             rowv.at[b], isem[b])
            pltpu.async_copy(ei_hbm.at[pl.ds(cb + c * K, K)],
                             colv.at[b], isem[b])

        def _i_wait(c, b):
            pltpu.make_async_copy(ei_hbm.at[pl.ds(rb + c * K, K)],
                                  rowv.at[b], isem[b]).wait()
            pltpu.make_async_copy(ei_hbm.at[pl.ds(cb + c * K, K)],
                                  colv.at[b], isem[b]).wait()

        def _per_table(fn):
            if len(tabs) == 1:
                fn(tabs[0])
            else:
                @pl.when(sc == 0)
                def _t0():
                    fn(tabs[0])

                @pl.when(sc == 1)
                def _t1():
                    fn(tabs[1])

        def _g_start(b):
            _per_table(lambda t: pltpu.async_copy(
                t.at[rowv.at[b]], rows.at[b], gsem[b]))

        def _g_wait(b):
            _per_table(lambda t: pltpu.make_async_copy(
                t.at[rowv.at[b]], rows.at[b], gsem[b]).wait())

        def _s_start(b):
            pltpu.async_copy(rows.at[b], acc.at[colv.at[b]], ssem[b],
                             add=True)

        def _s_wait(b):
            pltpu.make_async_copy(rows.at[b], acc.at[colv.at[b]],
                                  ssem[b]).wait()

        _per_table(lambda t: pltpu.sync_copy(t.at[pl.ds(r0, RPT)],
                                             acc.at[pl.ds(r0, RPT)]))

        @pl.when(tid == NT - 1)
        def _init_tail():
            _per_table(lambda t: pltpu.sync_copy(t.at[pl.ds(TAIL0, TAILN)],
                                                 acc.at[pl.ds(TAIL0, TAILN)]))

        plsc.subcore_barrier()
        _i_start(0, 0)

        def _group(p, carry):
            for b in range(NBUF):
                c = p * NBUF + b
                b1 = (b + 1) % NBUF
                bp = (b - 2) % NBUF
                _i_wait(c, b)
                _g_start(b)

                @pl.when(c >= NBUF - 1)
                def _free_next():
                    _s_wait(b1)

                @pl.when(c + 1 < NCH)
                def _prefetch():
                    _i_start(c + 1, b1)

                @pl.when(c >= 2)
                def _drain_prev():
                    _g_wait(bp)
                    _s_start(bp)
            return carry

        lax.fori_loop(0, NG, _group, 0)
        for t in range(REM):
            c = NG * NBUF + t
            b = t
            if t > 0:
                _s_wait(b)
                _i_start(c, b)
            _i_wait(c, b)
            _g_start(b)
            bp = (b - 2) % NBUF
            _g_wait(bp)
            _s_start(bp)
        for q in (NCH - 2, NCH - 1):
            _g_wait(q % NBUF)
            _s_start(q % NBUF)
        for q in range(NCH - NBUF + (1 if REM == 0 else 0), NCH):
            _s_wait(q % NBUF)
        plsc.subcore_barrier()
        pltpu.sync_copy(acc.at[pl.ds(r0, RPT)], out_hbm.at[sc, pl.ds(r0, RPT)])

        @pl.when(tid == NT - 1)
        def _out_tail():
            pltpu.sync_copy(acc.at[pl.ds(TAIL0, TAILN)],
                            out_hbm.at[sc, pl.ds(TAIL0, TAILN)])

    if edge_split:
        @functools.partial(
            pl.kernel, mesh=_MESH,
            out_type=jax.ShapeDtypeStruct((NC, N, H), jnp.float32),
            scratch_types=_SCRATCH,
            compiler_params=pltpu.CompilerParams(needs_layout_passes=False),
        )
        def _scatter(g_hbm, ei_hbm, out_hbm, rowv, colv, rows, acc, *sems):
            _scatter_body((g_hbm,), ei_hbm, out_hbm, rowv, colv, rows, acc,
                          sems)
    else:
        @functools.partial(
            pl.kernel, mesh=_MESH,
            out_type=jax.ShapeDtypeStruct((NC, N, H), jnp.float32),
            scratch_types=_SCRATCH,
            compiler_params=pltpu.CompilerParams(needs_layout_passes=False),
        )
        def _scatter(glo_hbm, ghi_hbm, ei_hbm, out_hbm,
                     rowv, colv, rows, acc, *sems):
            _scatter_body((glo_hbm, ghi_hbm), ei_hbm, out_hbm,
                          rowv, colv, rows, acc, sems)

    return _scatter


_scatter_l1 = _make_scatter(C_HID // 2, 80, 4, False)
_scatter_l2 = _make_scatter(C_OUT, 80, 4, True)


_BM = 2000


def _tc1_body(degp_ref, x_ref, w_ref, b_ref, glo_ref, ghi_ref, dis_ref):
    deg = degp_ref[pl.ds(0, N)] + 1.0
    for w in range(1, NW):
        deg = deg + degp_ref[pl.ds(w * N, N)]
    dis = lax.rsqrt(deg).reshape(-1, 1)
    h = lax.dot_general(x_ref[...], w_ref[...], (((1,), (1,)), ((), ())),
                        preferred_element_type=jnp.float32) + b_ref[...]
    g = h * dis
    glo_ref[...] = g[:, :C_HID // 2]
    ghi_ref[...] = g[:, C_HID // 2:]
    dis_ref[...] = dis


def _tc1(degp, x, W1, b1r):
    return pl.pallas_call(
        _tc1_body,
        grid=(1,),
        in_specs=[
            pl.BlockSpec((NW * N,), lambda i: (0,)),
            pl.BlockSpec((N, C_IN), lambda i: (0, 0)),
            pl.BlockSpec((C_HID, C_IN), lambda i: (0, 0)),
            pl.BlockSpec((1, C_HID), lambda i: (0, 0)),
        ],
        out_specs=[
            pl.BlockSpec((N, C_HID // 2), lambda i: (0, 0)),
            pl.BlockSpec((N, C_HID // 2), lambda i: (0, 0)),
            pl.BlockSpec((N, 1), lambda i: (0, 0)),
        ],
        out_shape=[
            jax.ShapeDtypeStruct((N, C_HID // 2), jnp.float32),
            jax.ShapeDtypeStruct((N, C_HID // 2), jnp.float32),
            jax.ShapeDtypeStruct((N, 1), jnp.float32),
        ],
    )(degp, x, W1, b1r)


def _tc2_body(s_ref, dis_ref, w_ref, b_ref, g_ref):
    dis = dis_ref[...]
    s = jnp.concatenate([s_ref[0], s_ref[1]], axis=1)
    u = jnp.maximum(s * dis, 0.0)
    h = lax.dot_general(u, w_ref[...], (((1,), (1,)), ((), ())),
                        preferred_element_type=jnp.float32) + b_ref[...]
    g_ref[...] = h * dis


def _tc2(s1, dis, W2, b2r):
    return pl.pallas_call(
        _tc2_body,
        grid=(N // _BM,),
        in_specs=[
            pl.BlockSpec((NC, _BM, C_HID // 2), lambda i: (0, i, 0)),
            pl.BlockSpec((_BM, 1), lambda i: (i, 0)),
            pl.BlockSpec((C_OUT, C_HID), lambda i: (0, 0)),
            pl.BlockSpec((1, C_OUT), lambda i: (0, 0)),
        ],
        out_specs=pl.BlockSpec((_BM, C_OUT), lambda i: (i, 0)),
        out_shape=jax.ShapeDtypeStruct((N, C_OUT), jnp.float32),
    )(s1, dis, W2, b2r)


def _tc3_body(p_ref, g_ref, dis_ref, o_ref):
    o_ref[...] = (p_ref[0] + p_ref[1] - g_ref[...]) * dis_ref[...]


def _tc3(p2, g2, dis):
    return pl.pallas_call(
        _tc3_body,
        grid=(N // _BM,),
        in_specs=[
            pl.BlockSpec((NC, _BM, C_OUT), lambda i: (0, i, 0)),
            pl.BlockSpec((_BM, C_OUT), lambda i: (i, 0)),
            pl.BlockSpec((_BM, 1), lambda i: (i, 0)),
        ],
        out_specs=pl.BlockSpec((_BM, C_OUT), lambda i: (i, 0)),
        out_shape=jax.ShapeDtypeStruct((N, C_OUT), jnp.float32),
    )(p2, g2, dis)


def kernel(x, edge_index, W1, b1, W2, b2):
    eif = edge_index.astype(jnp.int32).reshape(2 * E)
    degp = _deg_kernel(eif)
    glo, ghi, dis = _tc1(degp, x, W1, b1.reshape(1, -1))
    s1 = _scatter_l1(glo, ghi, eif)
    g2 = _tc2(s1, dis, W2, b2.reshape(1, -1))
    p2 = _scatter_l2(g2, eif)
    return _tc3(p2, g2, dis)


# speedup vs baseline: 1.7322x; 1.0074x over previous
import functools

import jax
import jax.numpy as jnp
from jax import lax
from jax.experimental import pallas as pl
from jax.experimental.pallas import tpu as pltpu
from jax.experimental.pallas import tpu_sc as plsc

N = 10000
E = 320000
C_IN = 128
C_HID = 256
C_OUT = 128

NC = 2
NT = 16
NW = NC * NT

_MESH = plsc.VectorSubcoreMesh(core_axis_name="c", subcore_axis_name="s")


EPW = E // NW


@functools.partial(
    pl.kernel, mesh=_MESH,
    out_type=jax.ShapeDtypeStruct((NW * N,), jnp.float32),
    scratch_types=[
        pltpu.VMEM((EPW,), jnp.int32),
        pltpu.VMEM((N,), jnp.float32),
    ],
    compiler_params=pltpu.CompilerParams(needs_layout_passes=False),
)
def _deg_kernel(ei_hbm, out_hbm, colv, degv):
    wid = lax.axis_index("s") * NC + lax.axis_index("c")
    pltpu.sync_copy(ei_hbm.at[pl.ds(E + wid * EPW, EPW)], colv)
    zeros = jnp.zeros((16,), jnp.float32)

    def _zero(i, carry):
        for j in range(5):
            degv[pl.ds(i * 80 + j * 16, 16)] = zeros
        return carry

    lax.fori_loop(0, N // 80, _zero, 0)
    ones = jnp.ones((16,), jnp.float32)

    def _count(i, carry):
        for j in range(5):
            idx = colv[pl.ds(i * 80 + j * 16, 16)]
            plsc.addupdate_scatter(degv, [idx], ones)
        return carry

    lax.fori_loop(0, EPW // 80, _count, 0)
    pltpu.sync_copy(degv, out_hbm.at[pl.ds(wid * N, N)])


def _make_scatter(H, K, NBUF, edge_split):
    EPT = (E // NC // NT) if edge_split else (E // NT)
    NCH = EPT // K
    NG = NCH // NBUF
    REM = NCH % NBUF
    RPT = 624
    TAIL0 = NT * RPT
    TAILN = N - TAIL0
    assert K % 8 == 0 and K <= 128 and NG >= 2

    _SCRATCH = [
        pltpu.VMEM((NBUF, K), jnp.int32),
        pltpu.VMEM((NBUF, K), jnp.int32),
        pltpu.VMEM((NBUF, K, H), jnp.float32),
        pltpu.VMEM_SHARED((N, H), jnp.float32),
    ] + [pltpu.SemaphoreType.DMA] * (3 * NBUF)

    def _scatter_body(tabs, ei_hbm, out_hbm, rowv, colv, rows, acc, sems):
        isem = sems[:NBUF]
        gsem = sems[NBUF:2 * NBUF]
        ssem = sems[2 * NBUF:]
        sc = lax.axis_index("c")
        tid = lax.axis_index("s")
        r0 = tid * RPT
        if edge_split:
            rb = sc * (E // NC) + tid * EPT
        else:
            rb = tid * EPT
        cb = E + rb

        def _i_start(c, b):
            pltpu.async_copy(ei_hbm.at[pl.ds(rb + c * K, K)],
                             rowv.at[b], isem[b])
            pltpu.async_copy(ei_hbm.at[pl.ds(cb + c * K, K)],
                             colv.at[b], isem[b])

        def _i_wait(c, b):
            pltpu.make_async_copy(ei_hbm.at[pl.ds(rb + c * K, K)],
                                  rowv.at[b], isem[b]).wait()
            pltpu.make_async_copy(ei_hbm.at[pl.ds(cb + c * K, K)],
                                  colv.at[b], isem[b]).wait()

        def _per_table(fn):
            if len(tabs) == 1:
                fn(tabs[0])
            else:
                @pl.when(sc == 0)
                def _t0():
                    fn(tabs[0])

                @pl.when(sc == 1)
                def _t1():
                    fn(tabs[1])

        def _g_start(b):
            _per_table(lambda t: pltpu.async_copy(
                t.at[rowv.at[b]], rows.at[b], gsem[b]))

        def _g_wait(b):
            _per_table(lambda t: pltpu.make_async_copy(
                t.at[rowv.at[b]], rows.at[b], gsem[b]).wait())

        def _s_start(b):
            pltpu.async_copy(rows.at[b], acc.at[colv.at[b]], ssem[b],
                             add=True)

        def _s_wait(b):
            pltpu.make_async_copy(rows.at[b], acc.at[colv.at[b]],
                                  ssem[b]).wait()

        _per_table(lambda t: pltpu.sync_copy(t.at[pl.ds(r0, RPT)],
                                             acc.at[pl.ds(r0, RPT)]))

        @pl.when(tid == NT - 1)
        def _init_tail():
            _per_table(lambda t: pltpu.sync_copy(t.at[pl.ds(TAIL0, TAILN)],
                                                 acc.at[pl.ds(TAIL0, TAILN)]))

        plsc.subcore_barrier()
        _i_start(0, 0)

        def _group(p, carry):
            for b in range(NBUF):
                c = p * NBUF + b
                b1 = (b + 1) % NBUF
                bp = (b - 2) % NBUF
                _i_wait(c, b)
                _g_start(b)

                @pl.when(c >= NBUF - 1)
                def _free_next():
                    _s_wait(b1)

                @pl.when(c + 1 < NCH)
                def _prefetch():
                    _i_start(c + 1, b1)

                @pl.when(c >= 2)
                def _drain_prev():
                    _g_wait(bp)
                    _s_start(bp)
            return carry

        lax.fori_loop(0, NG, _group, 0)
        for t in range(REM):
            c = NG * NBUF + t
            b = t
            if t > 0:
                _s_wait(b)
                _i_start(c, b)
            _i_wait(c, b)
            _g_start(b)
            bp = (b - 2) % NBUF
            _g_wait(bp)
            _s_start(bp)
        for q in (NCH - 2, NCH - 1):
            _g_wait(q % NBUF)
            _s_start(q % NBUF)
        for q in range(NCH - NBUF + (1 if REM == 0 else 0), NCH):
            _s_wait(q % NBUF)
        plsc.subcore_barrier()
        pltpu.sync_copy(acc.at[pl.ds(r0, RPT)], out_hbm.at[sc, pl.ds(r0, RPT)])

        @pl.when(tid == NT - 1)
        def _out_tail():
            pltpu.sync_copy(acc.at[pl.ds(TAIL0, TAILN)],
                            out_hbm.at[sc, pl.ds(TAIL0, TAILN)])

    if edge_split:
        @functools.partial(
            pl.kernel, mesh=_MESH,
            out_type=jax.ShapeDtypeStruct((NC, N, H), jnp.float32),
            scratch_types=_SCRATCH,
            compiler_params=pltpu.CompilerParams(needs_layout_passes=False),
        )
        def _scatter(g_hbm, ei_hbm, out_hbm, rowv, colv, rows, acc, *sems):
            _scatter_body((g_hbm,), ei_hbm, out_hbm, rowv, colv, rows, acc,
                          sems)
    else:
        @functools.partial(
            pl.kernel, mesh=_MESH,
            out_type=jax.ShapeDtypeStruct((NC, N, H), jnp.float32),
            scratch_types=_SCRATCH,
            compiler_params=pltpu.CompilerParams(needs_layout_passes=False),
        )
        def _scatter(glo_hbm, ghi_hbm, ei_hbm, out_hbm,
                     rowv, colv, rows, acc, *sems):
            _scatter_body((glo_hbm, ghi_hbm), ei_hbm, out_hbm,
                          rowv, colv, rows, acc, sems)

    return _scatter


_scatter_l1 = _make_scatter(C_HID // 2, 80, 4, False)
_scatter_l2 = _make_scatter(C_OUT, 80, 4, True)


_BM = 2000


def _tc1_body(degp_ref, x_ref, w_ref, b_ref, glo_ref, ghi_ref, dis_ref):
    deg = degp_ref[pl.ds(0, N)] + 1.0
    for w in range(1, NW):
        deg = deg + degp_ref[pl.ds(w * N, N)]
    dis = lax.rsqrt(deg).reshape(-1, 1)
    h = lax.dot_general(x_ref[...], w_ref[...], (((1,), (1,)), ((), ())),
                        preferred_element_type=jnp.float32) + b_ref[...]
    g = h * dis
    glo_ref[...] = g[:, :C_HID // 2]
    ghi_ref[...] = g[:, C_HID // 2:]
    dis_ref[...] = dis


def _tc1(degp, x, W1, b1r):
    return pl.pallas_call(
        _tc1_body,
        grid=(1,),
        in_specs=[
            pl.BlockSpec((NW * N,), lambda i: (0,)),
            pl.BlockSpec((N, C_IN), lambda i: (0, 0)),
            pl.BlockSpec((C_HID, C_IN), lambda i: (0, 0)),
            pl.BlockSpec((1, C_HID), lambda i: (0, 0)),
        ],
        out_specs=[
            pl.BlockSpec((N, C_HID // 2), lambda i: (0, 0)),
            pl.BlockSpec((N, C_HID // 2), lambda i: (0, 0)),
            pl.BlockSpec((N, 1), lambda i: (0, 0)),
        ],
        out_shape=[
            jax.ShapeDtypeStruct((N, C_HID // 2), jnp.float32),
            jax.ShapeDtypeStruct((N, C_HID // 2), jnp.float32),
            jax.ShapeDtypeStruct((N, 1), jnp.float32),
        ],
    )(degp, x, W1, b1r)


def _tc2_body(s_ref, dis_ref, w_ref, b_ref, g_ref):
    dis = dis_ref[...]
    s = jnp.concatenate([s_ref[0], s_ref[1]], axis=1)
    u = jnp.maximum(s * dis, 0.0)
    h = lax.dot_general(u, w_ref[...], (((1,), (1,)), ((), ())),
                        preferred_element_type=jnp.float32) + b_ref[...]
    g_ref[...] = h * dis


def _tc2(s1, dis, W2, b2r):
    return pl.pallas_call(
        _tc2_body,
        grid=(N // _BM,),
        in_specs=[
            pl.BlockSpec((NC, _BM, C_HID // 2), lambda i: (0, i, 0)),
            pl.BlockSpec((_BM, 1), lambda i: (i, 0)),
            pl.BlockSpec((C_OUT, C_HID), lambda i: (0, 0)),
            pl.BlockSpec((1, C_OUT), lambda i: (0, 0)),
        ],
        out_specs=pl.BlockSpec((_BM, C_OUT), lambda i: (i, 0)),
        out_shape=jax.ShapeDtypeStruct((N, C_OUT), jnp.float32),
    )(s1, dis, W2, b2r)


def _tc3_body(p_ref, g_ref, dis_ref, o_ref):
    o_ref[...] = (p_ref[0] + p_ref[1] - g_ref[...]) * dis_ref[...]


def _tc3(p2, g2, dis):
    return pl.pallas_call(
        _tc3_body,
        grid=(N // _BM,),
        in_specs=[
            pl.BlockSpec((NC, _BM, C_OUT), lambda i: (0, i, 0)),
            pl.BlockSpec((_BM, C_OUT), lambda i: (i, 0)),
            pl.BlockSpec((_BM, 1), lambda i: (i, 0)),
        ],
        out_specs=pl.BlockSpec((_BM, C_OUT), lambda i: (i, 0)),
        out_shape=jax.ShapeDtypeStruct((N, C_OUT), jnp.float32),
    )(p2, g2, dis)


def kernel(x, edge_index, W1, b1, W2, b2):
    eif = edge_index.astype(jnp.int32).reshape(2 * E)
    degp = _deg_kernel(eif)
    glo, ghi, dis = _tc1(degp, x, W1, b1.reshape(1, -1))
    s1 = _scatter_l1(glo, ghi, eif)
    g2 = _tc2(s1, dis, W2, b2.reshape(1, -1))
    p2 = _scatter_l2(g2, eif)
    return _tc3(p2, g2, dis)
